# baseline (speedup 1.0000x reference)
import functools

import jax
import jax.numpy as jnp
from jax import lax
from jax.experimental import pallas as pl
from jax.experimental.pallas import tpu as pltpu
from jax.experimental.pallas import tpu_sc as plsc

_BANK = 100000
_BATCH = 16384
_D = 128

_NC = 2
_NS = 16
_NW = _NC * _NS

_SC_ROWS = 16384
_PER_W = _SC_ROWS // _NW
_CHUNK = 256
_NBUF = 2
_SIZES = [_CHUNK] * (_PER_W // _CHUNK)

_TC_BLK = 8192
_NORM_BLKS = _BATCH // _TC_BLK
_SC_BLKS = _SC_ROWS // _TC_BLK
_COPY0 = (_BATCH + _SC_ROWS) // _TC_BLK
_NBLKS = (_BANK + _TC_BLK - 1) // _TC_BLK
_TC_GRID = _NORM_BLKS + (_NBLKS - _COPY0)


def _sc_copy_body(bank_hbm, out_hbm, buf, sem_in, sem_out):
    c = lax.axis_index("c")
    s = lax.axis_index("s")
    wid = s * _NC + c
    base = _BATCH + wid * _PER_W
    dst = wid * _PER_W
    offs = [sum(_SIZES[:i]) for i in range(len(_SIZES))]
    n = len(_SIZES)

    def start_in(i):
        return pltpu.async_copy(
            bank_hbm.at[pl.ds(base + offs[i], _SIZES[i])],
            buf.at[i % _NBUF, pl.ds(0, _SIZES[i])],
            sem_in[i % _NBUF],
        )

    def start_out(i):
        return pltpu.async_copy(
            buf.at[i % _NBUF, pl.ds(0, _SIZES[i])],
            out_hbm.at[pl.ds(dst + offs[i], _SIZES[i])],
            sem_out[i % _NBUF],
        )

    h_in = [None] * n
    h_out = [None] * n
    ahead = 1
    for j in range(min(ahead, n)):
        h_in[j] = start_in(j)
    for i in range(n):
        if i + ahead - _NBUF >= 0:
            h_out[i + ahead - _NBUF].wait()
        if i + ahead < n:
            h_in[i + ahead] = start_in(i + ahead)
        h_in[i].wait()
        h_out[i] = start_out(i)
    for i in range(max(0, n - _NBUF + ahead), n):
        h_out[i].wait()


@functools.partial(
    pl.kernel,
    out_type=jax.ShapeDtypeStruct((_SC_ROWS, _D), jnp.float32),
    mesh=plsc.VectorSubcoreMesh(core_axis_name="c", subcore_axis_name="s"),
    scratch_types=[
        pltpu.VMEM((_NBUF, _CHUNK, _D), jnp.float32),
        [pltpu.SemaphoreType.DMA] * _NBUF,
        [pltpu.SemaphoreType.DMA] * _NBUF,
    ],
)
def _sc_copy(bank_hbm, out_hbm, buf, sem_in, sem_out):
    _sc_copy_body(bank_hbm, out_hbm, buf, sem_in, sem_out)


def _tc_big_body(feats_ref, bank_ref, out_ref):
    g = pl.program_id(0)

    @pl.when(g < _NORM_BLKS)
    def _():
        x = feats_ref[...]
        n2 = jnp.sum(x * x, axis=1, keepdims=True)
        out_ref[...] = x * jax.lax.rsqrt(jnp.maximum(n2, 1e-24))

    @pl.when(g >= _NORM_BLKS)
    def _():
        out_ref[...] = bank_ref[...]


def _tc_merge_body(sc_ref, _big_ref, out_ref):
    out_ref[...] = sc_ref[...]


def kernel(feats, bank):
    sc_part = _sc_copy(bank)
    big = pl.pallas_call(
        _tc_big_body,
        grid=(_TC_GRID,),
        in_specs=[
            pl.BlockSpec(
                (_TC_BLK, _D), lambda g: (jnp.minimum(g, _NORM_BLKS - 1), 0)
            ),
            pl.BlockSpec(
                (_TC_BLK, _D),
                lambda g: (jnp.maximum(g + _COPY0 - _NORM_BLKS, _COPY0), 0),
            ),
        ],
        out_specs=pl.BlockSpec(
            (_TC_BLK, _D),
            lambda g: (
                jnp.where(g < _NORM_BLKS, g, g + _COPY0 - _NORM_BLKS),
                0,
            ),
        ),
        out_shape=jax.ShapeDtypeStruct((_BANK, _D), jnp.float32),
    )(feats, bank)
    return pl.pallas_call(
        _tc_merge_body,
        grid=(_SC_BLKS,),
        in_specs=[
            pl.BlockSpec((_TC_BLK, _D), lambda j: (j, 0)),
            pl.BlockSpec((8, _D), lambda j: (0, 0)),
        ],
        out_specs=pl.BlockSpec((_TC_BLK, _D), lambda j: (j + _NORM_BLKS, 0)),
        out_shape=jax.ShapeDtypeStruct((_BANK, _D), jnp.float32),
        input_output_aliases={1: 0},
    )(sc_part, big)
